# baseline (speedup 1.0000x reference)
import jax
import jax.numpy as jnp
from jax import lax
from jax.experimental import pallas as pl
from jax.experimental.pallas import tpu as pltpu
from jax.experimental.pallas import tpu_sc as plsc

M, D, B = 1000000, 32, 16384
NC, NS = 2, 16
UPT = B // NS
CH = 128
NCH = UPT // CH
DPC = D // NC
MAIN = 999936
TAIL = M - MAIN
COLS_A = 62464
COLS_B = MAIN - (NS - 1) * COLS_A


def _scatter_body(mem_t, idx2d, val_t, tail_in, out_t, tail_out,
                  idx_s, val_v0, val_v1, tail_v0, tail_v1,
                  row0, row1, lsem, ssem, vsem, csem):
    cid = lax.axis_index("c")
    sid = lax.axis_index("s")
    rows, vals, tails = [row0, row1], [val_v0, val_v1], [tail_v0, tail_v1]
    pltpu.sync_copy(idx2d.at[pl.ds(sid * (UPT // CH), UPT // CH)], idx_s)

    def load_pairs(k):
        d = cid * DPC + k
        buf, tv, vv = rows[k % 2], tails[k % 2], vals[k % 2]
        sa = sid * COLS_A
        sb = (NS - 1) * COLS_A
        main = (mem_t.at[d, pl.ds(sa, COLS_A)], buf.at[pl.ds(sa, COLS_A)])
        last = (mem_t.at[d, pl.ds(sb, COLS_B)], buf.at[pl.ds(sb, COLS_B)])
        return main, last, (tail_in.at[d], tv), (val_t.at[d, pl.ds(sid * UPT, UPT)], vv)

    def store_pairs(k):
        d = cid * DPC + k
        buf, tv = rows[k % 2], tails[k % 2]
        sa = sid * COLS_A
        sb = (NS - 1) * COLS_A
        main = (buf.at[pl.ds(sa, COLS_A)], out_t.at[d, pl.ds(sa, COLS_A)])
        last = (buf.at[pl.ds(sb, COLS_B)], out_t.at[d, pl.ds(sb, COLS_B)])
        return main, last, (tv, tail_out.at[d])

    def issue_load(k):
        main, last, tl, vl = load_pairs(k)
        pltpu.async_copy(vl[0], vl[1], vsem)

        @pl.when(sid < NS - 1)
        def _():
            pltpu.async_copy(main[0], main[1], lsem)

        @pl.when(sid == NS - 1)
        def _():
            pltpu.async_copy(last[0], last[1], lsem)
            pltpu.async_copy(tl[0], tl[1], lsem)

    def wait_load(k):
        main, last, tl, vl = load_pairs(k)
        buf, tv = rows[k % 2], tails[k % 2]
        pltpu.make_async_copy(vl[0], vl[1], vsem).wait()

        @pl.when(sid < NS - 1)
        def _():
            pltpu.make_async_copy(main[0], main[1], lsem).wait()

        @pl.when(sid == NS - 1)
        def _():
            pltpu.make_async_copy(last[0], last[1], lsem).wait()
            pltpu.make_async_copy(tl[0], tl[1], lsem).wait()
            pltpu.sync_copy(tv.at[pl.ds(0, TAIL)], buf.at[pl.ds(MAIN, TAIL)])

    def issue_store(k):
        main, last, ts = store_pairs(k)
        buf, tv = rows[k % 2], tails[k % 2]

        @pl.when(sid < NS - 1)
        def _():
            pltpu.async_copy(main[0], main[1], ssem)

        @pl.when(sid == NS - 1)
        def _():
            pltpu.sync_copy(buf.at[pl.ds(MAIN, TAIL)], tv.at[pl.ds(0, TAIL)])
            pltpu.async_copy(last[0], last[1], ssem)
            pltpu.async_copy(ts[0], ts[1], ssem)

    def wait_store(k):
        main, last, ts = store_pairs(k)

        @pl.when(sid < NS - 1)
        def _():
            pltpu.make_async_copy(main[0], main[1], ssem).wait()

        @pl.when(sid == NS - 1)
        def _():
            pltpu.make_async_copy(last[0], last[1], ssem).wait()
            pltpu.make_async_copy(ts[0], ts[1], ssem).wait()

    issue_load(0)
    for k in range(DPC):
        wait_load(k)
        plsc.subcore_barrier()
        vv = vals[k % 2]
        for j in range(NCH):
            pltpu.async_copy(vv.at[pl.ds(j * CH, CH)],
                             rows[k % 2].at[idx_s.at[j]], csem, add=True)
        for j in range(NCH):
            pltpu.make_async_copy(vv.at[pl.ds(j * CH, CH)],
                                  rows[k % 2].at[idx_s.at[j]], csem).wait()
        plsc.subcore_barrier()
        issue_store(k)
        if k + 1 < DPC:
            if k >= 1:
                wait_store(k - 1)
            issue_load(k + 1)
    wait_store(DPC - 2)
    wait_store(DPC - 1)


def kernel(mem, idx, val):
    run = pl.kernel(
        _scatter_body,
        out_type=(jax.ShapeDtypeStruct((D, M), jnp.float32),
                  jax.ShapeDtypeStruct((D, CH), jnp.float32)),
        mesh=plsc.VectorSubcoreMesh(core_axis_name="c", subcore_axis_name="s"),
        scratch_types=[
            pltpu.VMEM((UPT // CH, CH), jnp.int32),
            pltpu.VMEM((UPT,), jnp.float32),
            pltpu.VMEM((UPT,), jnp.float32),
            pltpu.VMEM((CH,), jnp.float32),
            pltpu.VMEM((CH,), jnp.float32),
            pltpu.VMEM_SHARED((M,), jnp.float32),
            pltpu.VMEM_SHARED((M,), jnp.float32),
            pltpu.SemaphoreType.DMA,
            pltpu.SemaphoreType.DMA,
            pltpu.SemaphoreType.DMA,
            pltpu.SemaphoreType.DMA,
        ],
    )
    tail_in = jnp.pad(mem[MAIN:].T, ((0, 0), (0, CH - TAIL)))
    out_t, tail_out = run(mem.T, idx.astype(jnp.int32).reshape(B // CH, CH),
                          val.T, tail_in)
    out = out_t.T
    return lax.dynamic_update_slice(out, tail_out[:, :TAIL].T, (MAIN, 0))


# speedup vs baseline: 1.0343x; 1.0343x over previous
import jax
import jax.numpy as jnp
from jax import lax
from jax.experimental import pallas as pl
from jax.experimental.pallas import tpu as pltpu
from jax.experimental.pallas import tpu_sc as plsc

M, D, B = 1000000, 32, 16384
NC, NS = 2, 16
UPT = B // NS
CH = 128
NCH = UPT // CH
DPC = D // NC
MAIN = 999936
TAIL = M - MAIN
COLS_A = 62464
COLS_B = MAIN - (NS - 1) * COLS_A


def _scatter_body(mem_t, idx2d, val_t, tail_in, out_t, tail_out,
                  idx_s, val_v0, val_v1, tail_v0, tail_v1,
                  row0, row1, lsem, ssem, vsem, csem):
    cid = lax.axis_index("c")
    sid = lax.axis_index("s")
    rows, vals, tails = [row0, row1], [val_v0, val_v1], [tail_v0, tail_v1]
    pltpu.sync_copy(idx2d.at[pl.ds(sid * (UPT // CH), UPT // CH)], idx_s)

    def load_pairs(k):
        d = cid * DPC + k
        buf, tv, vv = rows[k % 2], tails[k % 2], vals[k % 2]
        sa = sid * COLS_A
        sb = (NS - 1) * COLS_A
        main = (mem_t.at[d, pl.ds(sa, COLS_A)], buf.at[pl.ds(sa, COLS_A)])
        last = (mem_t.at[d, pl.ds(sb, COLS_B)], buf.at[pl.ds(sb, COLS_B)])
        return main, last, (tail_in.at[d], tv), (val_t.at[d, pl.ds(sid * UPT, UPT)], vv)

    def store_pairs(k):
        d = cid * DPC + k
        buf, tv = rows[k % 2], tails[k % 2]
        sa = sid * COLS_A
        sb = (NS - 1) * COLS_A
        main = (buf.at[pl.ds(sa, COLS_A)], out_t.at[d, pl.ds(sa, COLS_A)])
        last = (buf.at[pl.ds(sb, COLS_B)], out_t.at[d, pl.ds(sb, COLS_B)])
        return main, last, (tv, tail_out.at[d])

    def issue_load(k):
        main, last, tl, vl = load_pairs(k)
        pltpu.async_copy(vl[0], vl[1], vsem)

        @pl.when(sid < NS - 1)
        def _():
            pltpu.async_copy(main[0], main[1], lsem)

        @pl.when(sid == NS - 1)
        def _():
            pltpu.async_copy(last[0], last[1], lsem)
            pltpu.async_copy(tl[0], tl[1], lsem)

    def wait_load(k):
        main, last, tl, vl = load_pairs(k)
        buf, tv = rows[k % 2], tails[k % 2]
        pltpu.make_async_copy(vl[0], vl[1], vsem).wait()

        @pl.when(sid < NS - 1)
        def _():
            pltpu.make_async_copy(main[0], main[1], lsem).wait()

        @pl.when(sid == NS - 1)
        def _():
            pltpu.make_async_copy(last[0], last[1], lsem).wait()
            pltpu.make_async_copy(tl[0], tl[1], lsem).wait()
            pltpu.sync_copy(tv.at[pl.ds(0, TAIL)], buf.at[pl.ds(MAIN, TAIL)])

    def issue_store(k):
        main, last, ts = store_pairs(k)
        buf, tv = rows[k % 2], tails[k % 2]

        @pl.when(sid < NS - 1)
        def _():
            pltpu.async_copy(main[0], main[1], ssem)

        @pl.when(sid == NS - 1)
        def _():
            pltpu.sync_copy(buf.at[pl.ds(MAIN, TAIL)], tv.at[pl.ds(0, TAIL)])
            pltpu.async_copy(last[0], last[1], ssem)
            pltpu.async_copy(ts[0], ts[1], ssem)

    def wait_store(k):
        main, last, ts = store_pairs(k)

        @pl.when(sid < NS - 1)
        def _():
            pltpu.make_async_copy(main[0], main[1], ssem).wait()

        @pl.when(sid == NS - 1)
        def _():
            pltpu.make_async_copy(last[0], last[1], ssem).wait()
            pltpu.make_async_copy(ts[0], ts[1], ssem).wait()

    issue_load(0)
    for k in range(DPC):
        wait_load(k)
        plsc.subcore_barrier()
        if k + 1 < DPC:
            if k >= 1:
                wait_store(k - 1)
            issue_load(k + 1)
        vv = vals[k % 2]
        for j in range(NCH):
            pltpu.async_copy(vv.at[pl.ds(j * CH, CH)],
                             rows[k % 2].at[idx_s.at[j]], csem, add=True)
        for j in range(NCH):
            pltpu.make_async_copy(vv.at[pl.ds(j * CH, CH)],
                                  rows[k % 2].at[idx_s.at[j]], csem).wait()
        plsc.subcore_barrier()
        issue_store(k)
    wait_store(DPC - 2)
    wait_store(DPC - 1)


def kernel(mem, idx, val):
    run = pl.kernel(
        _scatter_body,
        out_type=(jax.ShapeDtypeStruct((D, M), jnp.float32),
                  jax.ShapeDtypeStruct((D, CH), jnp.float32)),
        mesh=plsc.VectorSubcoreMesh(core_axis_name="c", subcore_axis_name="s"),
        scratch_types=[
            pltpu.VMEM((UPT // CH, CH), jnp.int32),
            pltpu.VMEM((UPT,), jnp.float32),
            pltpu.VMEM((UPT,), jnp.float32),
            pltpu.VMEM((CH,), jnp.float32),
            pltpu.VMEM((CH,), jnp.float32),
            pltpu.VMEM_SHARED((M,), jnp.float32),
            pltpu.VMEM_SHARED((M,), jnp.float32),
            pltpu.SemaphoreType.DMA,
            pltpu.SemaphoreType.DMA,
            pltpu.SemaphoreType.DMA,
            pltpu.SemaphoreType.DMA,
        ],
    )
    tail_in = jnp.pad(mem[MAIN:].T, ((0, 0), (0, CH - TAIL)))
    out_t, tail_out = run(mem.T, idx.astype(jnp.int32).reshape(B // CH, CH),
                          val.T, tail_in)
    out = out_t.T
    return lax.dynamic_update_slice(out, tail_out[:, :TAIL].T, (MAIN, 0))


# speedup vs baseline: 1.0345x; 1.0001x over previous
import jax
import jax.numpy as jnp
from jax import lax
from jax.experimental import pallas as pl
from jax.experimental.pallas import tpu as pltpu
from jax.experimental.pallas import tpu_sc as plsc

M, D, B = 1000000, 32, 16384
NC, NS = 2, 16
UPT = B // NS
CH = 128
NCH = UPT // CH
DPC = D // NC
MAIN = 999936
TAIL = M - MAIN
COLS_A = 62592
COLS_B = MAIN - (NS - 1) * COLS_A


def _scatter_body(mem_t, idx2d, val_t, tail_in, out_t, tail_out,
                  idx_s, val_v0, val_v1, tail_v0, tail_v1,
                  row0, row1, lsem, ssem, vsem, csem):
    cid = lax.axis_index("c")
    sid = lax.axis_index("s")
    rows, vals, tails = [row0, row1], [val_v0, val_v1], [tail_v0, tail_v1]
    pltpu.sync_copy(idx2d.at[pl.ds(sid * (UPT // CH), UPT // CH)], idx_s)

    def load_pairs(k):
        d = cid * DPC + k
        buf, tv, vv = rows[k % 2], tails[k % 2], vals[k % 2]
        sa = sid * COLS_A
        sb = (NS - 1) * COLS_A
        main = (mem_t.at[d, pl.ds(sa, COLS_A)], buf.at[pl.ds(sa, COLS_A)])
        last = (mem_t.at[d, pl.ds(sb, COLS_B)], buf.at[pl.ds(sb, COLS_B)])
        return main, last, (tail_in.at[d], tv), (val_t.at[d, pl.ds(sid * UPT, UPT)], vv)

    def store_pairs(k):
        d = cid * DPC + k
        buf, tv = rows[k % 2], tails[k % 2]
        sa = sid * COLS_A
        sb = (NS - 1) * COLS_A
        main = (buf.at[pl.ds(sa, COLS_A)], out_t.at[d, pl.ds(sa, COLS_A)])
        last = (buf.at[pl.ds(sb, COLS_B)], out_t.at[d, pl.ds(sb, COLS_B)])
        return main, last, (tv, tail_out.at[d])

    def issue_load(k):
        main, last, tl, vl = load_pairs(k)
        pltpu.async_copy(vl[0], vl[1], vsem)

        @pl.when(sid < NS - 1)
        def _():
            pltpu.async_copy(main[0], main[1], lsem)

        @pl.when(sid == NS - 1)
        def _():
            pltpu.async_copy(last[0], last[1], lsem)
            pltpu.async_copy(tl[0], tl[1], lsem)

    def wait_load(k):
        main, last, tl, vl = load_pairs(k)
        buf, tv = rows[k % 2], tails[k % 2]
        pltpu.make_async_copy(vl[0], vl[1], vsem).wait()

        @pl.when(sid < NS - 1)
        def _():
            pltpu.make_async_copy(main[0], main[1], lsem).wait()

        @pl.when(sid == NS - 1)
        def _():
            pltpu.make_async_copy(last[0], last[1], lsem).wait()
            pltpu.make_async_copy(tl[0], tl[1], lsem).wait()
            pltpu.sync_copy(tv.at[pl.ds(0, TAIL)], buf.at[pl.ds(MAIN, TAIL)])

    def issue_store(k):
        main, last, ts = store_pairs(k)
        buf, tv = rows[k % 2], tails[k % 2]

        @pl.when(sid < NS - 1)
        def _():
            pltpu.async_copy(main[0], main[1], ssem)

        @pl.when(sid == NS - 1)
        def _():
            pltpu.sync_copy(buf.at[pl.ds(MAIN, TAIL)], tv.at[pl.ds(0, TAIL)])
            pltpu.async_copy(last[0], last[1], ssem)
            pltpu.async_copy(ts[0], ts[1], ssem)

    def wait_store(k):
        main, last, ts = store_pairs(k)

        @pl.when(sid < NS - 1)
        def _():
            pltpu.make_async_copy(main[0], main[1], ssem).wait()

        @pl.when(sid == NS - 1)
        def _():
            pltpu.make_async_copy(last[0], last[1], ssem).wait()
            pltpu.make_async_copy(ts[0], ts[1], ssem).wait()

    issue_load(0)
    for k in range(DPC):
        wait_load(k)
        plsc.subcore_barrier()
        if k + 1 < DPC:
            if k >= 1:
                wait_store(k - 1)
            issue_load(k + 1)
        vv = vals[k % 2]
        for j in range(NCH):
            pltpu.async_copy(vv.at[pl.ds(j * CH, CH)],
                             rows[k % 2].at[idx_s.at[j]], csem, add=True)
        for j in range(NCH):
            pltpu.make_async_copy(vv.at[pl.ds(j * CH, CH)],
                                  rows[k % 2].at[idx_s.at[j]], csem).wait()
        plsc.subcore_barrier()
        issue_store(k)
    wait_store(DPC - 2)
    wait_store(DPC - 1)


def kernel(mem, idx, val):
    run = pl.kernel(
        _scatter_body,
        out_type=(jax.ShapeDtypeStruct((D, M), jnp.float32),
                  jax.ShapeDtypeStruct((D, CH), jnp.float32)),
        mesh=plsc.VectorSubcoreMesh(core_axis_name="c", subcore_axis_name="s"),
        scratch_types=[
            pltpu.VMEM((UPT // CH, CH), jnp.int32),
            pltpu.VMEM((UPT,), jnp.float32),
            pltpu.VMEM((UPT,), jnp.float32),
            pltpu.VMEM((CH,), jnp.float32),
            pltpu.VMEM((CH,), jnp.float32),
            pltpu.VMEM_SHARED((M,), jnp.float32),
            pltpu.VMEM_SHARED((M,), jnp.float32),
            pltpu.SemaphoreType.DMA,
            pltpu.SemaphoreType.DMA,
            pltpu.SemaphoreType.DMA,
            pltpu.SemaphoreType.DMA,
        ],
    )
    tail_in = jnp.pad(mem[MAIN:].T, ((0, 0), (0, CH - TAIL)))
    out_t, tail_out = run(mem.T, idx.astype(jnp.int32).reshape(B // CH, CH),
                          val.T, tail_in)
    out = out_t.T
    return lax.dynamic_update_slice(out, tail_out[:, :TAIL].T, (MAIN, 0))
